# baseline (device time: 14959 ns/iter reference)
import jax
import jax.numpy as jnp
from jax import lax
from jax.experimental import pallas as pl
from jax.experimental.pallas import tpu as pltpu

C = 8


def kernel(x):
    m, n = x.shape
    mc = m // C

    def body(x_ref, out_ref, comm_ref, s1, r1, s2, r2):
        my_x = lax.axis_index("x")
        my_y = lax.axis_index("y")
        x_nbr = (1 - my_x, my_y)
        y_nbr = (my_x, 1 - my_y)

        for c in range(C):
            comm_ref[0, c, :, :] = x_ref[pl.ds(c * mc, mc), :].astype(
                jnp.bfloat16
            )

        barrier_sem = pltpu.get_barrier_semaphore()
        for nbr in (x_nbr, y_nbr):
            pl.semaphore_signal(
                barrier_sem, inc=1,
                device_id=nbr, device_id_type=pl.DeviceIdType.MESH,
            )
        pl.semaphore_wait(barrier_sem, 2)

        def hop1(c):
            return pltpu.make_async_remote_copy(
                src_ref=comm_ref.at[0, c],
                dst_ref=comm_ref.at[1, c],
                send_sem=s1.at[c],
                recv_sem=r1.at[c],
                device_id=x_nbr,
                device_id_type=pl.DeviceIdType.MESH,
            )

        def hop2(c):
            blk = out_ref.at[pl.ds(c * mc, mc), pl.ds(my_y * n, n)]
            return pltpu.make_async_remote_copy(
                src_ref=blk,
                dst_ref=blk,
                send_sem=s2.at[c],
                recv_sem=r2.at[c],
                device_id=y_nbr,
                device_id_type=pl.DeviceIdType.MESH,
            )

        for c in range(C):
            hop1(c).start()

        for c in range(C):
            hop1(c).wait_recv()
            out_ref[pl.ds(c * mc, mc), pl.ds(my_y * n, n)] = (
                comm_ref[0, c, :, :] + comm_ref[1, c, :, :]
            )
            hop2(c).start()

        for c in range(C):
            hop2(c).wait_recv()
        for c in range(C):
            hop1(c).wait_send()
            hop2(c).wait_send()

    return pl.pallas_call(
        body,
        out_shape=jax.ShapeDtypeStruct((m, 2 * n), jnp.bfloat16),
        in_specs=[pl.BlockSpec(memory_space=pltpu.VMEM)],
        out_specs=pl.BlockSpec(memory_space=pltpu.VMEM),
        scratch_shapes=[
            pltpu.VMEM((2, C, mc, n), jnp.bfloat16),
            pltpu.SemaphoreType.DMA((C,)),
            pltpu.SemaphoreType.DMA((C,)),
            pltpu.SemaphoreType.DMA((C,)),
            pltpu.SemaphoreType.DMA((C,)),
        ],
        compiler_params=pltpu.CompilerParams(collective_id=0),
    )(x)
